# Initial kernel scaffold; baseline (speedup 1.0000x reference)
import jax
import jax.numpy as jnp
from jax.experimental import pallas as pl


def kernel(data, emb, lin_W, att_i, att_j, att_em_i, att_em_j, gat_bias, bn_gamma, bn_beta, out_W, out_b):
    raise NotImplementedError("write your pallas kernel here")


# speedup vs baseline: 4.2785x; 4.2785x over previous
import functools

import jax
import jax.numpy as jnp
from jax.experimental import pallas as pl

N = 10000
D = 128
K = 16
B = 2
NP = 10240
RB = 256
CHUNK = 128
NCHUNK = NP // CHUNK
NEG = -3e38
BN_EPS = 1e-5


def _topk_body(w_ref, wT_ref, out_ref):
    wT = wT_ref[...]
    inv = jax.lax.rsqrt(jnp.sum(wT * wT, axis=0, keepdims=True))
    s = jnp.dot(w_ref[...], wT, preferred_element_type=jnp.float32)
    s = s * inv
    col = jax.lax.broadcasted_iota(jnp.int32, (RB, NP), 1)
    s = jnp.where(col < N, s, NEG)

    chunkcol = col // CHUNK
    m = jnp.max(s.reshape(RB, NCHUNK, CHUNK), axis=2)
    cid = jax.lax.broadcasted_iota(jnp.int32, (RB, NCHUNK), 1)
    lane = jax.lax.broadcasted_iota(jnp.int32, (RB, CHUNK), 1)

    mw = m
    vparts = []
    gparts = []
    for _ in range(K):
        gm = jnp.max(mw, axis=1)
        c = jnp.min(jnp.where(mw == gm[:, None], cid, NCHUNK), axis=1)
        ch = jnp.max(jnp.where(chunkcol == c[:, None], s, NEG)
                     .reshape(RB, NCHUNK, CHUNK), axis=1)
        vparts.append(ch)
        gparts.append(c[:, None] * CHUNK + lane)
        mw = jnp.where(cid == c[:, None], NEG, mw)

    v2 = jnp.concatenate(vparts, axis=1)
    g2 = jnp.concatenate(gparts, axis=1)

    outs = []
    for _ in range(K):
        gm = jnp.max(v2, axis=1)
        gi = jnp.min(jnp.where(v2 == gm[:, None], g2, jnp.int32(2**30)), axis=1)
        outs.append(gi[:, None])
        v2 = jnp.where(g2 == gi[:, None], NEG, v2)
    out_ref[...] = jnp.concatenate(outs, axis=1)


def _topk(emb):
    wpad = jnp.zeros((NP, D), jnp.float32).at[:N].set(emb)
    wTpad = wpad.T
    grid = NP // RB
    out = pl.pallas_call(
        _topk_body,
        grid=(grid,),
        in_specs=[
            pl.BlockSpec((RB, D), lambda i: (i, 0)),
            pl.BlockSpec((D, NP), lambda i: (0, 0)),
        ],
        out_specs=pl.BlockSpec((RB, K), lambda i: (i, 0)),
        out_shape=jax.ShapeDtypeStruct((NP, K), jnp.int32),
    )(wpad, wTpad)
    return out[:N]


XB = 400


def _xl_body(x_ref, e_ref, linT_ref, attx_ref, atte_ref, xl_ref, a_ref):
    xl = jnp.dot(x_ref[...], linT_ref[...], preferred_element_type=jnp.float32)
    xl_ref[...] = xl
    a = (jnp.dot(xl, attx_ref[...], preferred_element_type=jnp.float32)
         + jnp.dot(e_ref[...], atte_ref[...], preferred_element_type=jnp.float32))
    a_ref[...] = a


def _xl_scalars(x2, emb, lin_W, att_i, att_j, att_em_i, att_em_j):
    grid = (B * N) // XB
    nb = N // XB
    attx = jnp.stack([att_i, att_j], axis=1)
    atte = jnp.stack([att_em_i, att_em_j], axis=1)
    xl, a = pl.pallas_call(
        _xl_body,
        grid=(grid,),
        in_specs=[
            pl.BlockSpec((XB, D), lambda i: (i, 0)),
            pl.BlockSpec((XB, D), lambda i: (i % nb, 0)),
            pl.BlockSpec((D, D), lambda i: (0, 0)),
            pl.BlockSpec((D, 2), lambda i: (0, 0)),
            pl.BlockSpec((D, 2), lambda i: (0, 0)),
        ],
        out_specs=[
            pl.BlockSpec((XB, D), lambda i: (i, 0)),
            pl.BlockSpec((XB, 2), lambda i: (i, 0)),
        ],
        out_shape=[
            jax.ShapeDtypeStruct((B * N, D), jnp.float32),
            jax.ShapeDtypeStruct((B * N, 2), jnp.float32),
        ],
    )(x2, emb, lin_W.T, attx, atte)
    return xl, a[:, 0], a[:, 1]


def kernel(data, emb, lin_W, att_i, att_j, att_em_i, att_em_j, gat_bias, bn_gamma, bn_beta, out_W, out_b):
    nbr = _topk(emb)
    x2 = data.reshape(B * N, D)
    xl, a_i, a_j = _xl_scalars(x2, emb, lin_W, att_i, att_j, att_em_i, att_em_j)

    xl3 = xl.reshape(B, N, D)
    a_i3 = a_i.reshape(B, N)
    a_j3 = a_j.reshape(B, N)
    self_mask = nbr == jnp.arange(N)[:, None]
    a_j_nbr = a_j3[:, nbr]
    alpha_nbr = a_i3[:, :, None] + a_j_nbr
    alpha_self = a_i3 + a_j3
    alpha = jnp.concatenate([alpha_nbr, alpha_self[:, :, None]], axis=-1)
    alpha = jax.nn.leaky_relu(alpha, negative_slope=0.2)
    mask = jnp.concatenate([~self_mask, jnp.ones((N, 1), bool)], axis=-1)
    alpha = jnp.where(mask[None], alpha, -jnp.inf)
    amax = jnp.max(alpha, axis=-1, keepdims=True)
    ex = jnp.exp(alpha - amax)
    attn = ex / (jnp.sum(ex, axis=-1, keepdims=True) + 1e-16)
    x_nbr = xl3[:, nbr]
    gcn = (jnp.einsum('bnk,bnkd->bnd', attn[..., :K], x_nbr)
           + attn[..., K:K + 1] * xl3)
    gcn = gcn + gat_bias
    h = gcn * emb[None]
    h = h / jnp.sqrt(1.0 + BN_EPS) * bn_gamma + bn_beta
    h = jax.nn.relu(h)
    out = h @ out_W.T + out_b
    return out.reshape(-1, N)


# speedup vs baseline: 12.0628x; 2.8194x over previous
import functools

import jax
import jax.numpy as jnp
from jax import lax
from jax.experimental import pallas as pl
from jax.experimental.pallas import tpu as pltpu
from jax.experimental.pallas import tpu_sc as plsc

N = 10000
D = 128
K = 16
B = 2
NP = 10240
RB = 256
CHUNK = 128
NCHUNK = NP // CHUNK
NEG = -3e38
BN_EPS = 1e-5


def _topk_body(w_ref, wT_ref, out_ref):
    wT = wT_ref[...]
    inv = jax.lax.rsqrt(jnp.sum(wT * wT, axis=0, keepdims=True))
    s = jnp.dot(w_ref[...], wT, preferred_element_type=jnp.float32)
    s = s * inv
    col = jax.lax.broadcasted_iota(jnp.int32, (RB, NP), 1)
    s = jnp.where(col < N, s, NEG)

    m = jnp.max(s.reshape(RB, NCHUNK, CHUNK), axis=2)
    cid = jax.lax.broadcasted_iota(jnp.int32, (RB, NCHUNK), 1)
    lane = jax.lax.broadcasted_iota(jnp.int32, (RB, CHUNK), 1)

    mw = m
    ohparts = []
    gparts = []
    for _ in range(K):
        gm = jnp.max(mw, axis=1)
        c = jnp.min(jnp.where(mw == gm[:, None], cid, NCHUNK), axis=1)
        sel = cid == c[:, None]
        ohparts.append(jnp.where(sel, 1.0, 0.0))
        gparts.append(c[:, None] * CHUNK + lane)
        mw = jnp.where(sel, NEG, mw)

    oh = jnp.concatenate(ohparts, axis=1).reshape(RB, K, NCHUNK)
    ext = jax.lax.dot_general(oh, s.reshape(RB, NCHUNK, CHUNK),
                              (((2,), (1,)), ((0,), (0,))),
                              preferred_element_type=jnp.float32)
    v2 = ext.reshape(RB, K * CHUNK)
    g2 = jnp.concatenate(gparts, axis=1)

    outs = []
    for _ in range(K):
        gm = jnp.max(v2, axis=1)
        gi = jnp.min(jnp.where(v2 == gm[:, None], g2, jnp.int32(2**30)), axis=1)
        outs.append(gi[:, None])
        v2 = jnp.where(g2 == gi[:, None], NEG, v2)
    out_ref[...] = jnp.concatenate(outs, axis=1)


def _topk(emb):
    wpad = jnp.zeros((NP, D), jnp.float32).at[:N].set(emb)
    wTpad = wpad.T
    grid = NP // RB
    out = pl.pallas_call(
        _topk_body,
        grid=(grid,),
        in_specs=[
            pl.BlockSpec((RB, D), lambda i: (i, 0)),
            pl.BlockSpec((D, NP), lambda i: (0, 0)),
        ],
        out_specs=pl.BlockSpec((RB, K), lambda i: (i, 0)),
        out_shape=jax.ShapeDtypeStruct((NP, K), jnp.int32),
    )(wpad, wTpad)
    return out[:N]


XB = 400


def _xl_body(x_ref, e_ref, linT_ref, attx_ref, atte_ref, xl_ref, a_ref):
    xl = jnp.dot(x_ref[...], linT_ref[...], preferred_element_type=jnp.float32)
    xl_ref[...] = xl
    a = (jnp.dot(xl, attx_ref[...], preferred_element_type=jnp.float32)
         + jnp.dot(e_ref[...], atte_ref[...], preferred_element_type=jnp.float32))
    a_ref[...] = a


def _xl_scalars(x2, emb, lin_W, att_i, att_j, att_em_i, att_em_j):
    grid = (B * N) // XB
    nb = N // XB
    attx = jnp.stack([att_i, att_j], axis=1)
    atte = jnp.stack([att_em_i, att_em_j], axis=1)
    xl, a = pl.pallas_call(
        _xl_body,
        grid=(grid,),
        in_specs=[
            pl.BlockSpec((XB, D), lambda i: (i, 0)),
            pl.BlockSpec((XB, D), lambda i: (i % nb, 0)),
            pl.BlockSpec((D, D), lambda i: (0, 0)),
            pl.BlockSpec((D, 2), lambda i: (0, 0)),
            pl.BlockSpec((D, 2), lambda i: (0, 0)),
        ],
        out_specs=[
            pl.BlockSpec((XB, D), lambda i: (i, 0)),
            pl.BlockSpec((XB, 2), lambda i: (i, 0)),
        ],
        out_shape=[
            jax.ShapeDtypeStruct((B * N, D), jnp.float32),
            jax.ShapeDtypeStruct((B * N, 2), jnp.float32),
        ],
    )(x2, emb, lin_W.T, attx, atte)
    return xl, a[:, 0], a[:, 1]


NPWPAD = 640
NGRP = NPWPAD // 8


def _splat_i32(v):
    return jnp.zeros((16,), jnp.int32) + v


def _fwd_body(xl_hbm, nbr_hbm, ai_hbm, aj_hbm, emb_hbm, consts_hbm, out_hbm,
              nbr_v, aj_v, ai_v, consts_v, selfa_v, idx_v, rows_v,
              selfrows_v, embrows_v, attn_v, attnself_v, outv_v, sem):
    wid = lax.axis_index("s") * 2 + lax.axis_index("c")
    b = wid // 16
    base = (wid % 16) * NPWPAD
    bN = b * N

    pltpu.sync_copy(nbr_hbm.at[pl.ds(base, NPWPAD), :], nbr_v)
    pltpu.sync_copy(aj_hbm.at[b, :], aj_v.at[pl.ds(0, NP)])
    pltpu.sync_copy(ai_hbm.at[b, pl.ds(base, NPWPAD)], ai_v.at[pl.ds(0, NPWPAD)])
    pltpu.sync_copy(consts_hbm, consts_v)

    lane = lax.iota(jnp.int32, 16)
    m0 = lane == 0

    def group(g, _):
        nl0 = g * 8

        def idx_step(i, _c):
            nbrv = nbr_v[nl0 + i, :]
            nbrc = jnp.minimum(jnp.maximum(nbrv, 0), N - 1)
            idx_v[pl.ds(i * 16, 16)] = nbrc + bN
            return _c
        lax.fori_loop(0, 8, idx_step, None)
        copy = pltpu.make_async_copy(xl_hbm.at[idx_v], rows_v, sem)
        copy.start()
        selfrow0 = jnp.minimum(bN + base + nl0, B * N - 8)
        pltpu.sync_copy(xl_hbm.at[pl.ds(selfrow0, 8), :], selfrows_v)
        embrow0 = jnp.minimum(base + nl0, N - 8)
        pltpu.sync_copy(emb_hbm.at[pl.ds(embrow0, 8), :], embrows_v)

        ai16 = ai_v[pl.ds(nl0, 16)]
        ajs16 = aj_v[pl.ds(base + nl0, 16)]
        sa = ai16 + ajs16
        selfa_v[...] = jnp.where(sa > 0, sa, 0.2 * sa)

        def att_step(i, _c):
            nbrv = nbr_v[nl0 + i, :]
            nbrc = jnp.minimum(jnp.maximum(nbrv, 0), N - 1)
            ajv = plsc.load_gather(aj_v, [nbrc])
            aiv = plsc.load_gather(ai_v, [_splat_i32(nl0 + i)])
            al = aiv + ajv
            al = jnp.where(al > 0, al, 0.2 * al)
            selfmask = nbrv == _splat_i32(base + nl0 + i)
            al = jnp.where(selfmask, -1e30, al)
            sav = plsc.load_gather(selfa_v, [_splat_i32(i)])
            amax = jnp.max(jnp.maximum(al, sav))
            ex = jnp.where(selfmask, 0.0, jnp.exp(al - amax))
            exs_v = jnp.exp(sav - amax)
            denom = jnp.sum(ex) + jnp.max(exs_v) + 1e-16
            denom_v = jnp.zeros((16,), jnp.float32) + denom
            attn_v[pl.ds(i * 16, 16)] = ex / denom_v
            plsc.store_scatter(attnself_v, [_splat_i32(i)],
                               exs_v / denom_v, mask=m0)
            return _c
        lax.fori_loop(0, 8, att_step, None)

        copy.wait()

        def acc_step(i, _c):
            def k_step(k, acc):
                r = i * 16 + k
                w = plsc.load_gather(attn_v, [_splat_i32(r)])
                return tuple(acc[c] + w * rows_v[r, pl.ds(c * 16, 16)]
                             for c in range(8))
            acc = lax.fori_loop(
                0, 16, k_step,
                tuple(jnp.zeros((16,), jnp.float32) for _ in range(8)))
            ws = plsc.load_gather(attnself_v, [_splat_i32(i)])
            psum = jnp.zeros((16,), jnp.float32)
            for c in range(8):
                sl = pl.ds(c * 16, 16)
                hc = acc[c] + ws * selfrows_v[i, sl]
                hc = (hc + consts_v[0, sl]) * embrows_v[i, sl]
                hc = hc * consts_v[1, sl] + consts_v[2, sl]
                hc = jnp.maximum(hc, 0.0)
                psum = psum + hc * consts_v[3, sl]
            outn = jnp.sum(psum)
            plsc.store_scatter(outv_v, [_splat_i32(i)],
                               jnp.zeros((16,), jnp.float32) + outn,
                               mask=m0)
            return _c
        lax.fori_loop(0, 8, acc_step, None)

        outv_v[...] = outv_v[...] + consts_v[4, pl.ds(0, 16)]
        pltpu.sync_copy(outv_v.at[pl.ds(0, 8)],
                        out_hbm.at[b, pl.ds(base + nl0, 8)])
        return _

    lax.fori_loop(0, NGRP, group, None)


def _sc_forward(xl, nbr, a_i, a_j, emb, consts):
    mesh = plsc.VectorSubcoreMesh(core_axis_name="c", subcore_axis_name="s")
    f = pl.kernel(
        _fwd_body,
        out_type=jax.ShapeDtypeStruct((B, NP), jnp.float32),
        mesh=mesh,
        compiler_params=pltpu.CompilerParams(needs_layout_passes=False),
        scratch_types=[
            pltpu.VMEM((NPWPAD, K), jnp.int32),
            pltpu.VMEM((NP + 16,), jnp.float32),
            pltpu.VMEM((NPWPAD + 16,), jnp.float32),
            pltpu.VMEM((5, D), jnp.float32),
            pltpu.VMEM((16,), jnp.float32),
            pltpu.VMEM((128,), jnp.int32),
            pltpu.VMEM((128, D), jnp.float32),
            pltpu.VMEM((8, D), jnp.float32),
            pltpu.VMEM((8, D), jnp.float32),
            pltpu.VMEM((128,), jnp.float32),
            pltpu.VMEM((16,), jnp.float32),
            pltpu.VMEM((16,), jnp.float32),
            pltpu.SemaphoreType.DMA,
        ],
    )
    return f(xl, nbr, a_i, a_j, emb, consts)


def kernel(data, emb, lin_W, att_i, att_j, att_em_i, att_em_j, gat_bias, bn_gamma, bn_beta, out_W, out_b):
    nbr = _topk(emb)
    x2 = data.reshape(B * N, D)
    xl, a_i, a_j = _xl_scalars(x2, emb, lin_W, att_i, att_j, att_em_i, att_em_j)

    nbr_pad = jnp.zeros((NP, K), jnp.int32).at[:N].set(nbr)
    a_i2 = jnp.zeros((B, NP), jnp.float32).at[:, :N].set(a_i.reshape(B, N))
    a_j2 = jnp.zeros((B, NP), jnp.float32).at[:, :N].set(a_j.reshape(B, N))
    scale = bn_gamma / jnp.sqrt(1.0 + BN_EPS)
    consts = jnp.stack([
        gat_bias, scale, bn_beta, out_W[0],
        jnp.full((D,), out_b[0], jnp.float32),
    ])
    out = _sc_forward(xl, nbr_pad, a_i2, a_j2, emb, consts)
    return out[:, :N]
